# Initial kernel scaffold; baseline (speedup 1.0000x reference)
import jax
import jax.numpy as jnp
from jax.experimental import pallas as pl


def kernel(input_ids, table):
    raise NotImplementedError("write your pallas kernel here")


# speedup vs baseline: 1.6833x; 1.6833x over previous
import functools

import jax
import jax.numpy as jnp
from jax import lax
from jax.experimental import pallas as pl
from jax.experimental.pallas import tpu as pltpu
from jax.experimental.pallas import tpu_sc as plsc

NC = 2
NS = 16
NW = NC * NS

ROWS = 16384
COLS = 50
B = ROWS * COLS
D = 64
BPW = B // NW
CHUNK = 128
NCHUNK = BPW // CHUNK

_mesh = plsc.VectorSubcoreMesh(core_axis_name="c", subcore_axis_name="s")


@functools.partial(
    pl.kernel,
    mesh=_mesh,
    out_type=jax.ShapeDtypeStruct((B, D), jnp.float32),
    scratch_types=[
        pltpu.VMEM((NCHUNK, CHUNK), jnp.int32),
        pltpu.VMEM((CHUNK, D), jnp.float32),
        pltpu.SemaphoreType.DMA,
    ],
    compiler_params=pltpu.CompilerParams(use_tc_tiling_on_sc=False),
)
def _sc_gather(idx_hbm, table_hbm, out_hbm, idx_v, rows_v, gsem):
    wid = lax.axis_index("s") * NC + lax.axis_index("c")
    base = wid * BPW
    pltpu.sync_copy(idx_hbm.at[wid], idx_v)

    def body(j, carry):
        pltpu.async_copy(table_hbm.at[idx_v.at[j]], rows_v, gsem).wait()
        pltpu.sync_copy(rows_v, out_hbm.at[pl.ds(base + j * CHUNK, CHUNK)])
        return carry

    lax.fori_loop(0, NCHUNK, body, 0)


def kernel(input_ids, table):
    ids = input_ids.reshape(-1).astype(jnp.int32).reshape(NW, NCHUNK, CHUNK)
    out = _sc_gather(ids, table)
    return out.reshape(ROWS, COLS, D)


# speedup vs baseline: 1.8713x; 1.1117x over previous
import functools

import jax
import jax.numpy as jnp
from jax import lax
from jax.experimental import pallas as pl
from jax.experimental.pallas import tpu as pltpu
from jax.experimental.pallas import tpu_sc as plsc

NC = 2
NS = 16
NW = NC * NS

ROWS = 16384
COLS = 50
B = ROWS * COLS
D = 64
BPW = B // NW
CHUNK = 128
NCHUNK = BPW // CHUNK

_mesh = plsc.VectorSubcoreMesh(core_axis_name="c", subcore_axis_name="s")


SUPER = 4
SW = SUPER * CHUNK
NSUPER = NCHUNK // SUPER
GRP = NSUPER // 2


@functools.partial(
    pl.kernel,
    mesh=_mesh,
    out_type=jax.ShapeDtypeStruct((B, D), jnp.float32),
    scratch_types=[
        pltpu.VMEM((NCHUNK, CHUNK), jnp.int32),
        pltpu.VMEM((2, SW, D), jnp.float32),
        pltpu.SemaphoreType.DMA,
        pltpu.SemaphoreType.DMA,
        pltpu.SemaphoreType.DMA,
        pltpu.SemaphoreType.DMA,
    ],
    compiler_params=pltpu.CompilerParams(use_tc_tiling_on_sc=False),
)
def _sc_gather(idx_hbm, table_hbm, out_hbm, idx_v, big_v, g0, g1, s0, s1):
    wid = lax.axis_index("s") * NC + lax.axis_index("c")
    base = wid * BPW
    gsem = (g0, g1)
    ssem = (s0, s1)
    pltpu.sync_copy(idx_hbm.at[wid], idx_v)

    def fire4(s, p):
        for b in range(SUPER):
            j = s * SUPER + b
            pltpu.async_copy(
                table_hbm.at[idx_v.at[j]],
                big_v.at[p].at[pl.ds(b * CHUNK, CHUNK)],
                gsem[p],
            )

    def drain4(s, p):
        pltpu.make_async_copy(
            out_hbm.at[pl.ds(base + s * SW, SW)], big_v.at[p], gsem[p]
        ).wait()

    def store_start(s, p):
        pltpu.async_copy(
            big_v.at[p], out_hbm.at[pl.ds(base + s * SW, SW)], ssem[p]
        )

    def store_wait(s, p):
        pltpu.make_async_copy(
            big_v.at[p], out_hbm.at[pl.ds(base + s * SW, SW)], ssem[p]
        ).wait()

    fire4(0, 0)

    def body(g, carry):
        s = 2 * g
        drain4(s, 0)
        store_start(s, 0)

        @pl.when(g > 0)
        def _():
            store_wait(s - 1, 1)

        fire4(s + 1, 1)
        drain4(s + 1, 1)
        store_start(s + 1, 1)

        @pl.when(g < GRP - 1)
        def _():
            store_wait(s, 0)
            fire4(s + 2, 0)

        return carry

    lax.fori_loop(0, GRP, body, 0)
    store_wait(NSUPER - 2, 0)
    store_wait(NSUPER - 1, 1)


def kernel(input_ids, table):
    ids = input_ids.reshape(-1).astype(jnp.int32).reshape(NW, NCHUNK, CHUNK)
    out = _sc_gather(ids, table)
    return out.reshape(ROWS, COLS, D)
